# baseline (device time: 42975 ns/iter reference)
import jax
import jax.numpy as jnp
from jax import lax
from jax.experimental import pallas as pl
from jax.experimental.pallas import tpu as pltpu

N_DEV = 8
N_LAYERS = 3
N_STEPS = 3
B = 64
H = 2048
D = 1024

CHUNKS = (
    (0, 512, (1, 3, 4)),
    (512, 512, (3, 4, 1)),
    (1024, 512, (4, 1, 3)),
    (1536, 256, (1, 3, 4)),
    (1792, 128, (3, 4, 1)),
    (1920, 128, (4, 1, 3)),
)
N_CHUNKS = len(CHUNKS)
N_SLOTS = N_LAYERS * N_STEPS


def kernel(x, Win0, Wout0, Win1, Wout1, Win2, Wout2):
    b, d_in = x.shape

    def body(
        x_ref,
        win0_ref,
        wout0_ref,
        win1_ref,
        wout1_ref,
        win2_ref,
        wout2_ref,
        out_ref,
        win_v0,
        win_v1,
        win_v2,
        wout_v0,
        wout_v1,
        wout_v2,
        *scratch,
    ):
        my = lax.axis_index("i")
        accs = list(scratch[:N_CHUNKS])
        recvs = list(scratch[N_CHUNKS : 2 * N_CHUNKS])
        copy_sems, send_sems, recv_sems = scratch[2 * N_CHUNKS :]
        win_hbm = [win0_ref, win1_ref, win2_ref]
        wout_hbm = [wout0_ref, wout1_ref, wout2_ref]
        win_v = [win_v0, win_v1, win_v2]
        wout_v = [wout_v0, wout_v1, wout_v2]

        k = 0
        win_chunk_handles = {}
        win_whole_handles = {}
        wout_handles = {}
        for p in range(N_CHUNKS):
            off, w, _ = CHUNKS[p]
            cp = pltpu.make_async_copy(
                win_hbm[0].at[:, off : off + w],
                win_v[0].at[:, off : off + w],
                copy_sems.at[k],
            )
            cp.start()
            win_chunk_handles[(0, p)] = cp
            k += 1
        for layer in range(N_LAYERS):
            if layer > 0:
                cp = pltpu.make_async_copy(
                    win_hbm[layer], win_v[layer], copy_sems.at[k]
                )
                cp.start()
                win_whole_handles[layer] = cp
                k += 1
            cp = pltpu.make_async_copy(
                wout_hbm[layer], wout_v[layer], copy_sems.at[k]
            )
            cp.start()
            wout_handles[layer] = cp
            k += 1

        barrier_sem = pltpu.get_barrier_semaphore()
        for mask in (1, 3, 4):
            pl.semaphore_signal(
                barrier_sem,
                inc=1,
                device_id=(my ^ mask,),
                device_id_type=pl.DeviceIdType.MESH,
            )
        pl.semaphore_wait(barrier_sem, N_STEPS)

        def start(p, s, layer):
            idx = layer * N_STEPS + s
            rdma = pltpu.make_async_remote_copy(
                src_ref=accs[p],
                dst_ref=recvs[p].at[idx],
                send_sem=send_sems.at[p, idx],
                recv_sem=recv_sems.at[p, idx],
                device_id=(my ^ CHUNKS[p][2][s],),
                device_id_type=pl.DeviceIdType.MESH,
            )
            rdma.start()
            return rdma

        x_val = x_ref[...]
        for layer in range(N_LAYERS):
            handles = [None] * N_CHUNKS
            if layer > 0:
                win_whole_handles[layer].wait()
            for p in range(N_CHUNKS):
                off, w, _ = CHUNKS[p]
                if layer == 0:
                    win_chunk_handles[(0, p)].wait()
                accs[p][...] = jnp.dot(
                    x_val,
                    win_v[layer][:, off : off + w],
                    preferred_element_type=jnp.float32,
                )
                handles[p] = start(p, 0, layer)
            x_next = jnp.zeros((b, d_in), jnp.float32)
            wout_waited = False
            for s in range(N_STEPS):
                for p in range(N_CHUNKS):
                    idx = layer * N_STEPS + s
                    handles[p].wait()
                    if s < N_STEPS - 1:
                        accs[p][...] = accs[p][...] + recvs[p][idx]
                        handles[p] = start(p, s + 1, layer)
                    else:
                        off, w, _ = CHUNKS[p]
                        if not wout_waited:
                            wout_handles[layer].wait()
                            wout_waited = True
                        h_p = jnp.maximum(
                            accs[p][...] + recvs[p][idx], 0.0
                        )
                        x_next = x_next + jnp.dot(
                            h_p,
                            wout_v[layer][off : off + w, :],
                            preferred_element_type=jnp.float32,
                        )
            x_val = x_next
        out_ref[...] = x_val

    return pl.pallas_call(
        body,
        out_shape=jax.ShapeDtypeStruct((b, d_in), jnp.float32),
        in_specs=[pl.BlockSpec(memory_space=pltpu.VMEM)]
        + [pl.BlockSpec(memory_space=pltpu.MemorySpace.HBM)] * 6,
        out_specs=pl.BlockSpec(memory_space=pltpu.VMEM),
        scratch_shapes=[
            pltpu.VMEM((D, H), jnp.float32),
            pltpu.VMEM((D, H), jnp.float32),
            pltpu.VMEM((D, H), jnp.float32),
            pltpu.VMEM((H, D), jnp.float32),
            pltpu.VMEM((H, D), jnp.float32),
            pltpu.VMEM((H, D), jnp.float32),
            *[pltpu.VMEM((B, c[1]), jnp.float32) for c in CHUNKS],
            *[pltpu.VMEM((N_SLOTS, B, c[1]), jnp.float32) for c in CHUNKS],
            pltpu.SemaphoreType.DMA((N_CHUNKS + 5,)),
            pltpu.SemaphoreType.DMA((N_CHUNKS, N_SLOTS)),
            pltpu.SemaphoreType.DMA((N_CHUNKS, N_SLOTS)),
        ],
        compiler_params=pltpu.CompilerParams(
            collective_id=0, vmem_limit_bytes=100 * 1024 * 1024
        ),
    )(x, Win0, Wout0, Win1, Wout1, Win2, Wout2)


# device time: 35157 ns/iter; 1.2224x vs baseline; 1.2224x over previous
import jax
import jax.numpy as jnp
from jax import lax
from jax.experimental import pallas as pl
from jax.experimental.pallas import tpu as pltpu

N_DEV = 8
N_LAYERS = 3
N_STEPS = 3
B = 64
H = 2048
D = 1024

CHUNKS = (
    (0, 384, (1, 3, 4)),
    (384, 384, (3, 4, 1)),
    (768, 384, (4, 1, 3)),
    (1152, 384, (1, 3, 4)),
    (1536, 256, (3, 4, 1)),
    (1792, 256, (4, 1, 3)),
)
N_CHUNKS = len(CHUNKS)
N_SLOTS = N_LAYERS * N_STEPS


def kernel(x, Win0, Wout0, Win1, Wout1, Win2, Wout2):
    b, d_in = x.shape

    def body(
        x_ref,
        win0_ref,
        wout0_ref,
        win1_ref,
        wout1_ref,
        win2_ref,
        wout2_ref,
        out_ref,
        win_v0,
        win_v1,
        win_v2,
        wout_v0,
        wout_v1,
        wout_v2,
        *scratch,
    ):
        my = lax.axis_index("i")
        accs = list(scratch[:N_CHUNKS])
        recvs = list(scratch[N_CHUNKS : 2 * N_CHUNKS])
        copy_sems, send_sems, recv_sems = scratch[2 * N_CHUNKS :]
        win_hbm = [win0_ref, win1_ref, win2_ref]
        wout_hbm = [wout0_ref, wout1_ref, wout2_ref]
        win_v = [win_v0, win_v1, win_v2]
        wout_v = [wout_v0, wout_v1, wout_v2]

        k = 0
        win_chunk_handles = {}
        win_whole_handles = {}
        wout_handles = {}
        for p in range(N_CHUNKS):
            off, w, _ = CHUNKS[p]
            cp = pltpu.make_async_copy(
                win_hbm[0].at[:, off : off + w],
                win_v[0].at[:, off : off + w],
                copy_sems.at[k],
            )
            cp.start()
            win_chunk_handles[(0, p)] = cp
            k += 1
        for layer in range(N_LAYERS):
            if layer > 0:
                cp = pltpu.make_async_copy(
                    win_hbm[layer], win_v[layer], copy_sems.at[k]
                )
                cp.start()
                win_whole_handles[layer] = cp
                k += 1
            cp = pltpu.make_async_copy(
                wout_hbm[layer], wout_v[layer], copy_sems.at[k]
            )
            cp.start()
            wout_handles[layer] = cp
            k += 1

        barrier_sem = pltpu.get_barrier_semaphore()
        for mask in (1, 3, 4):
            pl.semaphore_signal(
                barrier_sem,
                inc=1,
                device_id=(my ^ mask,),
                device_id_type=pl.DeviceIdType.MESH,
            )
        pl.semaphore_wait(barrier_sem, N_STEPS)

        def start(p, s, layer):
            idx = layer * N_STEPS + s
            rdma = pltpu.make_async_remote_copy(
                src_ref=accs[p],
                dst_ref=recvs[p].at[idx],
                send_sem=send_sems.at[p, idx],
                recv_sem=recv_sems.at[p, idx],
                device_id=(my ^ CHUNKS[p][2][s],),
                device_id_type=pl.DeviceIdType.MESH,
            )
            rdma.start()
            return rdma

        x_val = x_ref[...]
        for layer in range(N_LAYERS):
            handles = [None] * N_CHUNKS
            if layer > 0:
                win_whole_handles[layer].wait()
            for p in range(N_CHUNKS):
                off, w, _ = CHUNKS[p]
                if layer == 0:
                    win_chunk_handles[(0, p)].wait()
                accs[p][...] = jnp.dot(
                    x_val,
                    win_v[layer][:, off : off + w],
                    preferred_element_type=jnp.float32,
                ).astype(jnp.bfloat16)
                handles[p] = start(p, 0, layer)
            x_next = jnp.zeros((b, d_in), jnp.float32)
            wout_waited = False
            for s in range(N_STEPS):
                for p in range(N_CHUNKS):
                    idx = layer * N_STEPS + s
                    handles[p].wait()
                    if s < N_STEPS - 1:
                        accs[p][...] = accs[p][...] + recvs[p][idx]
                        handles[p] = start(p, s + 1, layer)
                    else:
                        off, w, _ = CHUNKS[p]
                        if not wout_waited:
                            wout_handles[layer].wait()
                            wout_waited = True
                        h_p = jnp.maximum(
                            accs[p][...] + recvs[p][idx],
                            jnp.bfloat16(0.0),
                        ).astype(jnp.float32)
                        x_next = x_next + jnp.dot(
                            h_p,
                            wout_v[layer][off : off + w, :],
                            preferred_element_type=jnp.float32,
                        )
            x_val = x_next
        out_ref[...] = x_val

    return pl.pallas_call(
        body,
        out_shape=jax.ShapeDtypeStruct((b, d_in), jnp.float32),
        in_specs=[pl.BlockSpec(memory_space=pltpu.VMEM)]
        + [pl.BlockSpec(memory_space=pltpu.MemorySpace.HBM)] * 6,
        out_specs=pl.BlockSpec(memory_space=pltpu.VMEM),
        scratch_shapes=[
            pltpu.VMEM((D, H), jnp.float32),
            pltpu.VMEM((D, H), jnp.float32),
            pltpu.VMEM((D, H), jnp.float32),
            pltpu.VMEM((H, D), jnp.float32),
            pltpu.VMEM((H, D), jnp.float32),
            pltpu.VMEM((H, D), jnp.float32),
            *[pltpu.VMEM((B, c[1]), jnp.bfloat16) for c in CHUNKS],
            *[pltpu.VMEM((N_SLOTS, B, c[1]), jnp.bfloat16) for c in CHUNKS],
            pltpu.SemaphoreType.DMA((N_CHUNKS + 5,)),
            pltpu.SemaphoreType.DMA((N_CHUNKS, N_SLOTS)),
            pltpu.SemaphoreType.DMA((N_CHUNKS, N_SLOTS)),
        ],
        compiler_params=pltpu.CompilerParams(
            collective_id=0, vmem_limit_bytes=100 * 1024 * 1024
        ),
    )(x, Win0, Wout0, Win1, Wout1, Win2, Wout2)


# device time: 35105 ns/iter; 1.2242x vs baseline; 1.0015x over previous
import jax
import jax.numpy as jnp
from jax import lax
from jax.experimental import pallas as pl
from jax.experimental.pallas import tpu as pltpu

N_DEV = 8
N_LAYERS = 3
N_STEPS = 3
B = 64
H = 2048
D = 1024

CHUNKS = (
    (0, 384, (1, 3, 4)),
    (384, 384, (3, 4, 1)),
    (768, 384, (4, 1, 3)),
    (1152, 384, (1, 3, 4)),
    (1536, 256, (3, 4, 1)),
    (1792, 256, (4, 1, 3)),
)
N_CHUNKS = len(CHUNKS)
N_SLOTS = N_LAYERS * N_STEPS


def kernel(x, Win0, Wout0, Win1, Wout1, Win2, Wout2):
    b, d_in = x.shape

    def body(
        x_ref,
        win0_ref,
        wout0_ref,
        win1_ref,
        wout1_ref,
        win2_ref,
        wout2_ref,
        out_ref,
        win_v0,
        win_v1,
        win_v2,
        wout_v0,
        wout_v1,
        wout_v2,
        *scratch,
    ):
        my = lax.axis_index("i")
        accs = list(scratch[:N_CHUNKS])
        recvs = list(scratch[N_CHUNKS : 2 * N_CHUNKS])
        copy_sems, send_sems, recv_sems = scratch[2 * N_CHUNKS :]
        win_hbm = [win0_ref, win1_ref, win2_ref]
        wout_hbm = [wout0_ref, wout1_ref, wout2_ref]
        win_v = [win_v0, win_v1, win_v2]
        wout_v = [wout_v0, wout_v1, wout_v2]

        k = 0
        win_chunk_handles = {}
        win_whole_handles = {}
        wout_handles = {}
        for p in range(N_CHUNKS):
            off, w, _ = CHUNKS[p]
            cp = pltpu.make_async_copy(
                win_hbm[0].at[:, off : off + w],
                win_v[0].at[:, off : off + w],
                copy_sems.at[k],
            )
            cp.start()
            win_chunk_handles[(0, p)] = cp
            k += 1
        for layer in range(N_LAYERS):
            if layer > 0:
                cp = pltpu.make_async_copy(
                    win_hbm[layer], win_v[layer], copy_sems.at[k]
                )
                cp.start()
                win_whole_handles[layer] = cp
                k += 1
            cp = pltpu.make_async_copy(
                wout_hbm[layer], wout_v[layer], copy_sems.at[k]
            )
            cp.start()
            wout_handles[layer] = cp
            k += 1

        barrier_sem = pltpu.get_barrier_semaphore()
        for mask in (1, 3, 4):
            pl.semaphore_signal(
                barrier_sem,
                inc=1,
                device_id=(my ^ mask,),
                device_id_type=pl.DeviceIdType.MESH,
            )

        def start(p, s, layer):
            idx = layer * N_STEPS + s
            rdma = pltpu.make_async_remote_copy(
                src_ref=accs[p],
                dst_ref=recvs[p].at[idx],
                send_sem=send_sems.at[p, idx],
                recv_sem=recv_sems.at[p, idx],
                device_id=(my ^ CHUNKS[p][2][s],),
                device_id_type=pl.DeviceIdType.MESH,
            )
            rdma.start()
            return rdma

        x_val = x_ref[...]
        for layer in range(N_LAYERS):
            handles = [None] * N_CHUNKS
            if layer > 0:
                win_whole_handles[layer].wait()
            for p in range(N_CHUNKS):
                off, w, _ = CHUNKS[p]
                if layer == 0:
                    win_chunk_handles[(0, p)].wait()
                accs[p][...] = jnp.dot(
                    x_val,
                    win_v[layer][:, off : off + w],
                    preferred_element_type=jnp.float32,
                ).astype(jnp.bfloat16)
                if layer == 0 and p == 0:
                    pl.semaphore_wait(barrier_sem, N_STEPS)
                handles[p] = start(p, 0, layer)
            x_next = jnp.zeros((b, d_in), jnp.float32)
            wout_waited = False
            for s in range(N_STEPS):
                for p in range(N_CHUNKS):
                    idx = layer * N_STEPS + s
                    handles[p].wait()
                    if s < N_STEPS - 1:
                        accs[p][...] = accs[p][...] + recvs[p][idx]
                        handles[p] = start(p, s + 1, layer)
                    else:
                        off, w, _ = CHUNKS[p]
                        if not wout_waited:
                            wout_handles[layer].wait()
                            wout_waited = True
                        h_p = jnp.maximum(
                            accs[p][...] + recvs[p][idx],
                            jnp.bfloat16(0.0),
                        ).astype(jnp.float32)
                        x_next = x_next + jnp.dot(
                            h_p,
                            wout_v[layer][off : off + w, :],
                            preferred_element_type=jnp.float32,
                        )
            x_val = x_next
        out_ref[...] = x_val

    return pl.pallas_call(
        body,
        out_shape=jax.ShapeDtypeStruct((b, d_in), jnp.float32),
        in_specs=[pl.BlockSpec(memory_space=pltpu.VMEM)]
        + [pl.BlockSpec(memory_space=pltpu.MemorySpace.HBM)] * 6,
        out_specs=pl.BlockSpec(memory_space=pltpu.VMEM),
        scratch_shapes=[
            pltpu.VMEM((D, H), jnp.float32),
            pltpu.VMEM((D, H), jnp.float32),
            pltpu.VMEM((D, H), jnp.float32),
            pltpu.VMEM((H, D), jnp.float32),
            pltpu.VMEM((H, D), jnp.float32),
            pltpu.VMEM((H, D), jnp.float32),
            *[pltpu.VMEM((B, c[1]), jnp.bfloat16) for c in CHUNKS],
            *[pltpu.VMEM((N_SLOTS, B, c[1]), jnp.bfloat16) for c in CHUNKS],
            pltpu.SemaphoreType.DMA((N_CHUNKS + 5,)),
            pltpu.SemaphoreType.DMA((N_CHUNKS, N_SLOTS)),
            pltpu.SemaphoreType.DMA((N_CHUNKS, N_SLOTS)),
        ],
        compiler_params=pltpu.CompilerParams(
            collective_id=0, vmem_limit_bytes=100 * 1024 * 1024
        ),
    )(x, Win0, Wout0, Win1, Wout1, Win2, Wout2)
